# baseline (speedup 1.0000x reference)
import functools

import jax
import jax.numpy as jnp
from jax import lax
from jax.experimental import pallas as pl
from jax.experimental.pallas import tpu as pltpu
from jax.experimental.pallas import tpu_sc as plsc

_B = 16384
_P = 21
_E = 64
_N = _B * _P
_NW = 32
_SW = _B // _NW
_TW = _SW // 128
_PW = _N // _NW
_CH = 96
_NCH = _PW // _CH


def _sc_body(x_hbm, emb_u_hbm, emb_v_hbm, out_hbm,
             xv, uidx, vidx, u_rows0, v_rows0, u_rows1, v_rows1, s_buf,
             out_v, sem_u0, sem_v0, sem_u1, sem_v1, sem_i):
    c = lax.axis_index("c")
    s = lax.axis_index("s")
    wid = s * 2 + c
    lanes = jnp.arange(16, dtype=jnp.int32)
    lanes16 = lanes * 16

    pltpu.async_copy(x_hbm.at[:, pl.ds(wid * _TW, _TW)], xv, sem_i).wait()

    @pl.loop(0, _P * _TW * 8, unroll=4)
    def deint(i):
        j = i // (_TW * 8)
        rem = i - j * (_TW * 8)
        tc = rem // 8
        l0 = (rem - tc * 8) * 16
        p0 = j * _SW + tc * 128 + l0
        uidx[pl.ds(p0, 16)] = xv[j, tc, 0, pl.ds(l0, 16)]
        vidx[pl.ds(p0, 16)] = xv[j, tc, 1, pl.ds(l0, 16)]

    bufs = ((u_rows0, v_rows0, sem_u0, sem_v0),
            (u_rows1, v_rows1, sem_u1, sem_v1))

    def start_gather(ci, slot):
        ur, vr, su, sv = bufs[slot]
        pltpu.async_copy(emb_u_hbm.at[uidx.at[pl.ds(ci * _CH, _CH)]], ur, su)
        pltpu.async_copy(emb_v_hbm.at[vidx.at[pl.ds(ci * _CH, _CH)]], vr, sv)

    def wait_gather(ci, slot):
        ur, vr, su, sv = bufs[slot]
        pltpu.make_async_copy(
            emb_u_hbm.at[uidx.at[pl.ds(ci * _CH, _CH)]], ur, su).wait()
        pltpu.make_async_copy(
            emb_v_hbm.at[vidx.at[pl.ds(ci * _CH, _CH)]], vr, sv).wait()

    def compute(ci, slot):
        ur, vr, _, _ = bufs[slot]

        @pl.loop(0, _CH, unroll=8)
        def kstep(k):
            acc = (ur[k, pl.ds(0, 16)] * vr[k, pl.ds(0, 16)]
                   + ur[k, pl.ds(16, 16)] * vr[k, pl.ds(16, 16)]
                   + ur[k, pl.ds(32, 16)] * vr[k, pl.ds(32, 16)]
                   + ur[k, pl.ds(48, 16)] * vr[k, pl.ds(48, 16)])
            s_buf[pl.ds(k * 16, 16)] = acc

        for g in range(_CH // 16):
            acc = plsc.load_gather(s_buf, [lanes16 + g * 256])
            for j in range(1, 16):
                acc = acc + plsc.load_gather(s_buf, [lanes16 + (g * 256 + j)])
            q0 = ci * _CH + g * 16
            pj = q0 // _SW
            sl = q0 - pj * _SW
            out_v[pj, pl.ds(sl, 16)] = acc

    start_gather(0, 0)

    @pl.loop(0, _NCH // 2)
    def group(g):
        a = 2 * g
        b = a + 1
        wait_gather(a, 0)
        start_gather(b, 1)
        compute(a, 0)
        wait_gather(b, 1)
        start_gather(jnp.minimum(a + 2, _NCH - 1), 0)
        compute(b, 1)

    wait_gather(_NCH - 1, 0)
    pltpu.sync_copy(out_v, out_hbm.at[:, pl.ds(wid * _SW, _SW)])


@jax.jit
def _sc_dot(xview, emb_u, emb_v):
    mesh = plsc.VectorSubcoreMesh(core_axis_name="c", subcore_axis_name="s")
    f = functools.partial(
        pl.kernel,
        out_type=jax.ShapeDtypeStruct((_P, _B), jnp.float32),
        mesh=mesh,
        compiler_params=pltpu.CompilerParams(
            needs_layout_passes=False, use_tc_tiling_on_sc=False),
        scratch_types=[
            pltpu.VMEM((_P, _TW, 2, 128), jnp.int32),
            pltpu.VMEM((_PW,), jnp.int32),
            pltpu.VMEM((_PW,), jnp.int32),
            pltpu.VMEM((_CH, _E), jnp.float32),
            pltpu.VMEM((_CH, _E), jnp.float32),
            pltpu.VMEM((_CH, _E), jnp.float32),
            pltpu.VMEM((_CH, _E), jnp.float32),
            pltpu.VMEM((_CH * 16,), jnp.float32),
            pltpu.VMEM((_P, _SW), jnp.float32),
            pltpu.SemaphoreType.DMA,
            pltpu.SemaphoreType.DMA,
            pltpu.SemaphoreType.DMA,
            pltpu.SemaphoreType.DMA,
            pltpu.SemaphoreType.DMA,
        ],
    )(_sc_body)
    return f(xview, emb_u, emb_v)


def kernel(x, emb_u, emb_v):
    xview = jnp.transpose(
        jnp.transpose(x, (1, 2, 0)).reshape(_P, 2, 128, 128), (0, 2, 1, 3))
    out_t = _sc_dot(xview, emb_u, emb_v)
    return out_t.T


# speedup vs baseline: 1.0028x; 1.0028x over previous
import functools

import jax
import jax.numpy as jnp
from jax import lax
from jax.experimental import pallas as pl
from jax.experimental.pallas import tpu as pltpu
from jax.experimental.pallas import tpu_sc as plsc

_B = 16384
_P = 21
_E = 64
_N = _B * _P
_NW = 32
_SW = _B // _NW
_TW = _SW // 128
_PW = _N // _NW
_CH = 96
_NCH = _PW // _CH


def _sc_body(x_hbm, emb_u_hbm, emb_v_hbm, out_hbm,
             xv, uidx, vidx, u_rows0, v_rows0, u_rows1, v_rows1, s_buf,
             out_v, sem_u0, sem_v0, sem_u1, sem_v1, sem_i):
    c = lax.axis_index("c")
    s = lax.axis_index("s")
    wid = s * 2 + c
    lanes = jnp.arange(16, dtype=jnp.int32)
    lanes16 = lanes * 16

    pltpu.async_copy(x_hbm.at[:, pl.ds(wid * _TW, _TW)], xv, sem_i).wait()

    @pl.loop(0, _P * _TW * 8, unroll=4)
    def deint(i):
        j = i // (_TW * 8)
        rem = i - j * (_TW * 8)
        tc = rem // 8
        l0 = (rem - tc * 8) * 16
        p0 = j * _SW + tc * 128 + l0
        uidx[pl.ds(p0, 16)] = xv[j, tc, 0, pl.ds(l0, 16)]
        vidx[pl.ds(p0, 16)] = xv[j, tc, 1, pl.ds(l0, 16)]

    bufs = ((u_rows0, v_rows0, sem_u0, sem_v0),
            (u_rows1, v_rows1, sem_u1, sem_v1))

    def start_gather(ci, slot):
        ur, vr, su, sv = bufs[slot]
        pltpu.async_copy(emb_u_hbm.at[uidx.at[pl.ds(ci * _CH, _CH)]], ur, su)
        pltpu.async_copy(emb_v_hbm.at[vidx.at[pl.ds(ci * _CH, _CH)]], vr, sv)

    def wait_gather(ci, slot):
        ur, vr, su, sv = bufs[slot]
        pltpu.make_async_copy(
            emb_u_hbm.at[uidx.at[pl.ds(ci * _CH, _CH)]], ur, su).wait()
        pltpu.make_async_copy(
            emb_v_hbm.at[vidx.at[pl.ds(ci * _CH, _CH)]], vr, sv).wait()

    def compute(ci, slot):
        ur, vr, _, _ = bufs[slot]

        @pl.loop(0, _CH, unroll=8)
        def kstep(k):
            acc = (ur[k, pl.ds(0, 16)] * vr[k, pl.ds(0, 16)]
                   + ur[k, pl.ds(16, 16)] * vr[k, pl.ds(16, 16)]
                   + ur[k, pl.ds(32, 16)] * vr[k, pl.ds(32, 16)]
                   + ur[k, pl.ds(48, 16)] * vr[k, pl.ds(48, 16)])
            s_buf[pl.ds(k * 16, 16)] = acc

        for g in range(_CH // 16):
            acc0 = plsc.load_gather(s_buf, [lanes16 + g * 256])
            acc1 = plsc.load_gather(s_buf, [lanes16 + (g * 256 + 1)])
            for j in range(2, 16, 2):
                acc0 = acc0 + plsc.load_gather(s_buf, [lanes16 + (g * 256 + j)])
                acc1 = acc1 + plsc.load_gather(
                    s_buf, [lanes16 + (g * 256 + j + 1)])
            acc = acc0 + acc1
            q0 = ci * _CH + g * 16
            pj = q0 // _SW
            sl = q0 - pj * _SW
            out_v[pj, pl.ds(sl, 16)] = acc

    start_gather(0, 0)

    @pl.loop(0, _NCH // 2)
    def group(g):
        a = 2 * g
        b = a + 1
        wait_gather(a, 0)
        start_gather(b, 1)
        compute(a, 0)
        wait_gather(b, 1)
        start_gather(jnp.minimum(a + 2, _NCH - 1), 0)
        compute(b, 1)

    wait_gather(_NCH - 1, 0)
    pltpu.sync_copy(out_v, out_hbm.at[:, pl.ds(wid * _SW, _SW)])


@jax.jit
def _sc_dot(xview, emb_u, emb_v):
    mesh = plsc.VectorSubcoreMesh(core_axis_name="c", subcore_axis_name="s")
    f = functools.partial(
        pl.kernel,
        out_type=jax.ShapeDtypeStruct((_P, _B), jnp.float32),
        mesh=mesh,
        compiler_params=pltpu.CompilerParams(
            needs_layout_passes=False, use_tc_tiling_on_sc=False),
        scratch_types=[
            pltpu.VMEM((_P, _TW, 2, 128), jnp.int32),
            pltpu.VMEM((_PW,), jnp.int32),
            pltpu.VMEM((_PW,), jnp.int32),
            pltpu.VMEM((_CH, _E), jnp.float32),
            pltpu.VMEM((_CH, _E), jnp.float32),
            pltpu.VMEM((_CH, _E), jnp.float32),
            pltpu.VMEM((_CH, _E), jnp.float32),
            pltpu.VMEM((_CH * 16,), jnp.float32),
            pltpu.VMEM((_P, _SW), jnp.float32),
            pltpu.SemaphoreType.DMA,
            pltpu.SemaphoreType.DMA,
            pltpu.SemaphoreType.DMA,
            pltpu.SemaphoreType.DMA,
            pltpu.SemaphoreType.DMA,
        ],
    )(_sc_body)
    return f(xview, emb_u, emb_v)


def kernel(x, emb_u, emb_v):
    xview = jnp.transpose(
        jnp.transpose(x, (1, 2, 0)).reshape(_P, 2, 128, 128), (0, 2, 1, 3))
    out_t = _sc_dot(xview, emb_u, emb_v)
    return out_t.T
